# baseline (speedup 1.0000x reference)
import functools

import jax
import jax.numpy as jnp
from jax import lax
from jax.experimental import pallas as pl
from jax.experimental.pallas import tpu as pltpu
from jax.experimental.pallas import tpu_sc as plsc

HIDDEN = 2048
INTER = 1408
N_ROUTED = 8
TOP_K = 2
BM = 256
NB_SH = 4096 // BM


_SC_NC = 2
_SC_NS = 16
_SC_NW = _SC_NC * _SC_NS


def _sc_gather(sorted_ids, x_flat):
    MAXR = sorted_ids.shape[0]
    BPW = MAXR // _SC_NW
    CH = 16
    NCH = BPW // CH
    mesh = plsc.VectorSubcoreMesh(core_axis_name="c", subcore_axis_name="s")

    @functools.partial(
        pl.kernel, mesh=mesh,
        out_type=jax.ShapeDtypeStruct((MAXR, HIDDEN), jnp.float32),
        scratch_types=[
            pltpu.VMEM((BPW,), jnp.int32),
            pltpu.VMEM((CH, HIDDEN), jnp.float32),
            pltpu.VMEM((CH, HIDDEN), jnp.float32),
            pltpu.SemaphoreType.DMA,
            pltpu.SemaphoreType.DMA,
        ],
    )
    def gather_k(ids_hbm, x_hbm, out_hbm, idx_v, rows0, rows1, sem0, sem1):
        wid = lax.axis_index("s") * _SC_NC + lax.axis_index("c")
        base = wid * BPW
        pltpu.sync_copy(ids_hbm.at[pl.ds(base, BPW)], idx_v)
        bufs = (rows0, rows1)
        sems = (sem0, sem1)
        handles = [None] * NCH
        handles[0] = pltpu.async_copy(
            x_hbm.at[idx_v.at[pl.ds(0, CH)]], bufs[0], sems[0])
        for i in range(NCH):
            if i + 1 < NCH:
                handles[i + 1] = pltpu.async_copy(
                    x_hbm.at[idx_v.at[pl.ds((i + 1) * CH, CH)]],
                    bufs[(i + 1) % 2], sems[(i + 1) % 2])
            handles[i].wait()
            pltpu.sync_copy(bufs[i % 2],
                            out_hbm.at[pl.ds(base + i * CH, CH)])

    return gather_k(sorted_ids, x_flat)


def _ffn(xb, wg, wu, wd):
    gp = jax.lax.dot_general(xb, wg, (((1,), (0,)), ((), ())),
                             preferred_element_type=jnp.float32)
    up = jax.lax.dot_general(xb, wu, (((1,), (0,)), ((), ())),
                             preferred_element_type=jnp.float32)
    h = (jax.nn.silu(gp) * up).astype(jnp.bfloat16)
    return jax.lax.dot_general(h, wd, (((1,), (0,)), ((), ())),
                               preferred_element_type=jnp.float32
                               ).astype(jnp.bfloat16)


def _mm_body(be_ref, na_ref, x_ref, xd_ref, wg_ref, wu_ref, wd_ref, out_ref):
    g = pl.program_id(0)

    @pl.when(g < NB_SH)
    def _():
        out_ref[...] = _ffn(x_ref[...].astype(jnp.bfloat16),
                            wg_ref[0], wu_ref[0], wd_ref[0])

    @pl.when((g >= NB_SH) & (g < na_ref[0]))
    def _():
        out_ref[...] = _ffn(xd_ref[...].astype(jnp.bfloat16),
                            wg_ref[0], wu_ref[0], wd_ref[0])


def _grouped_ffn(x_flat, xd, block_e, num_active, Wg, Wu, Wd, nb):
    T = x_flat.shape[0]
    R = xd.shape[0]
    nbr = R // BM

    def clampg(g, na_ref):
        return jnp.minimum(g, na_ref[0] - 1)

    grid_spec = pltpu.PrefetchScalarGridSpec(
        num_scalar_prefetch=2,
        grid=(nb,),
        in_specs=[
            pl.BlockSpec((BM, HIDDEN),
                         lambda g, be, na: (jnp.minimum(g, NB_SH - 1), 0)),
            pl.BlockSpec((BM, HIDDEN),
                         lambda g, be, na: (
                             jnp.clip(g - NB_SH, 0, nbr - 1), 0)),
            pl.BlockSpec((1, HIDDEN, INTER),
                         lambda g, be, na: (be[clampg(g, na)], 0, 0)),
            pl.BlockSpec((1, HIDDEN, INTER),
                         lambda g, be, na: (be[clampg(g, na)], 0, 0)),
            pl.BlockSpec((1, INTER, HIDDEN),
                         lambda g, be, na: (be[clampg(g, na)], 0, 0)),
        ],
        out_specs=pl.BlockSpec((BM, HIDDEN),
                               lambda g, be, na: (clampg(g, na), 0)),
    )
    return pl.pallas_call(
        _mm_body,
        grid_spec=grid_spec,
        out_shape=jax.ShapeDtypeStruct((T + R, HIDDEN), jnp.bfloat16),
    )(block_e, num_active, x_flat, xd, Wg, Wu, Wd)


def kernel(x, gate_w, expert_bias, shared_Wg, shared_Wu, shared_Wd,
           routed_Wg, routed_Wu, routed_Wd):
    Bc, Sc, H = x.shape
    T = Bc * Sc
    A = T * TOP_K
    x_flat = x.reshape(T, H)

    Wg9 = jnp.concatenate([routed_Wg, shared_Wg]).astype(jnp.bfloat16)
    Wu9 = jnp.concatenate([routed_Wu, shared_Wu]).astype(jnp.bfloat16)
    Wd9 = jnp.concatenate([routed_Wd, shared_Wd]).astype(jnp.bfloat16)

    x_fp32 = x.astype(jnp.float32)
    gate_logits = x_fp32 @ gate_w.astype(jnp.float32).T
    scores = jax.nn.sigmoid(gate_logits)
    tie = jnp.arange(N_ROUTED, dtype=jnp.float32) * 1e-6
    scores_for_routing = scores + expert_bias.astype(jnp.float32) + tie
    lanes = jnp.arange(N_ROUTED, dtype=jnp.int32)
    m1 = jnp.max(scores_for_routing, axis=-1, keepdims=True)
    i1 = jnp.min(jnp.where(scores_for_routing == m1, lanes, N_ROUTED),
                 axis=-1, keepdims=True)
    sr2 = jnp.where(lanes == i1, jnp.float32(-1e30), scores_for_routing)
    m2 = jnp.max(sr2, axis=-1, keepdims=True)
    i2 = jnp.min(jnp.where(sr2 == m2, lanes, N_ROUTED),
                 axis=-1, keepdims=True)
    top_k_indices = jnp.concatenate([i1, i2], axis=-1)
    s1 = jnp.sum(jnp.where(lanes == i1, scores, 0.0), axis=-1, keepdims=True)
    s2 = jnp.sum(jnp.where(lanes == i2, scores, 0.0), axis=-1, keepdims=True)
    denom = jnp.maximum(s1 + s2, 1e-9)
    top_k_weights = (jnp.concatenate([s1, s2], axis=-1) / denom).astype(x.dtype)

    idx2 = top_k_indices.reshape(T, TOP_K)
    w2 = top_k_weights.reshape(T, TOP_K)

    e_flat = idx2.reshape(A)
    tok_ids = (jnp.arange(A, dtype=jnp.int32) // TOP_K)
    onehot = (e_flat[:, None] == jnp.arange(N_ROUTED)[None, :]).astype(jnp.int32)
    counts = jnp.sum(onehot, axis=0)
    rank = jnp.sum(jnp.where(onehot == 1, jnp.cumsum(onehot, axis=0) - 1, 0),
                   axis=1)
    padded = ((counts + BM - 1) // BM) * BM
    offs = jnp.concatenate([jnp.zeros((1,), jnp.int32),
                            jnp.cumsum(padded).astype(jnp.int32)])
    pos = offs[e_flat] + rank

    MAXR = A + N_ROUTED * BM
    sorted_ids = jnp.zeros((MAXR,), jnp.int32).at[pos].set(tok_ids)

    NB_RT = MAXR // BM
    NB = NB_SH + NB_RT
    block_start = jnp.arange(NB_RT, dtype=jnp.int32) * BM
    block_e = jnp.clip(
        jnp.sum(block_start[:, None] >= offs[None, 1:], axis=1), 0, N_ROUTED - 1
    ).astype(jnp.int32)
    full_be = jnp.concatenate(
        [jnp.full((NB_SH,), N_ROUTED, jnp.int32), block_e])
    num_active = (NB_SH + offs[N_ROUTED] // BM).reshape(1).astype(jnp.int32)

    xd = jnp.take(x_flat, sorted_ids, axis=0)

    y = _grouped_ffn(x_flat, xd, full_be, num_active, Wg9, Wu9, Wd9, NB)

    p = pos.reshape(T, TOP_K)
    out_flat = (y[:T]
                + w2[:, 0:1] * jnp.take(y, T + p[:, 0], axis=0)
                + w2[:, 1:2] * jnp.take(y, T + p[:, 1], axis=0))
    return out_flat.reshape(Bc, Sc, H), top_k_indices


# speedup vs baseline: 1.1338x; 1.1338x over previous
import functools

import jax
import jax.numpy as jnp
from jax import lax
from jax.experimental import pallas as pl
from jax.experimental.pallas import tpu as pltpu
from jax.experimental.pallas import tpu_sc as plsc

HIDDEN = 2048
INTER = 1408
N_ROUTED = 8
TOP_K = 2
BM = 256
NB_SH = 4096 // BM


_SC_NC = 2
_SC_NS = 16
_SC_NW = _SC_NC * _SC_NS


def _sc_gather(sorted_ids, x_flat):
    MAXR = sorted_ids.shape[0]
    BPW = MAXR // _SC_NW
    CH = 16
    NCH = BPW // CH
    mesh = plsc.VectorSubcoreMesh(core_axis_name="c", subcore_axis_name="s")

    @functools.partial(
        pl.kernel, mesh=mesh,
        out_type=jax.ShapeDtypeStruct((MAXR, HIDDEN), jnp.float32),
        scratch_types=[
            pltpu.VMEM((BPW,), jnp.int32),
            pltpu.VMEM((CH, HIDDEN), jnp.float32),
            pltpu.VMEM((CH, HIDDEN), jnp.float32),
            pltpu.SemaphoreType.DMA,
            pltpu.SemaphoreType.DMA,
        ],
    )
    def gather_k(ids_hbm, x_hbm, out_hbm, idx_v, rows0, rows1, sem0, sem1):
        wid = lax.axis_index("s") * _SC_NC + lax.axis_index("c")
        base = wid * BPW
        pltpu.sync_copy(ids_hbm.at[pl.ds(base, BPW)], idx_v)
        bufs = (rows0, rows1)
        sems = (sem0, sem1)
        handles = [None] * NCH
        handles[0] = pltpu.async_copy(
            x_hbm.at[idx_v.at[pl.ds(0, CH)]], bufs[0], sems[0])
        for i in range(NCH):
            if i + 1 < NCH:
                handles[i + 1] = pltpu.async_copy(
                    x_hbm.at[idx_v.at[pl.ds((i + 1) * CH, CH)]],
                    bufs[(i + 1) % 2], sems[(i + 1) % 2])
            handles[i].wait()
            pltpu.sync_copy(bufs[i % 2],
                            out_hbm.at[pl.ds(base + i * CH, CH)])

    return gather_k(sorted_ids, x_flat)


def _ffn(xb, wg, wu, wd):
    gp = jax.lax.dot_general(xb, wg, (((1,), (0,)), ((), ())),
                             preferred_element_type=jnp.float32)
    up = jax.lax.dot_general(xb, wu, (((1,), (0,)), ((), ())),
                             preferred_element_type=jnp.float32)
    h = (jax.nn.silu(gp) * up).astype(jnp.bfloat16)
    return jax.lax.dot_general(h, wd, (((1,), (0,)), ((), ())),
                               preferred_element_type=jnp.float32)


def _mm_body(be_ref, na_ref, x_ref, xd_ref, wg_ref, wu_ref, wd_ref, out_ref):
    g = pl.program_id(0)

    @pl.when(g < NB_SH)
    def _():
        out_ref[...] = _ffn(x_ref[...].astype(jnp.bfloat16),
                            wg_ref[0], wu_ref[0], wd_ref[0])

    @pl.when((g >= NB_SH) & (g < na_ref[0]))
    def _():
        out_ref[...] = _ffn(xd_ref[...].astype(jnp.bfloat16),
                            wg_ref[0], wu_ref[0], wd_ref[0])


def _grouped_ffn(x_flat, xd, block_e, num_active, Wg, Wu, Wd, nb):
    T = x_flat.shape[0]
    R = xd.shape[0]
    nbr = R // BM

    def clampg(g, na_ref):
        return jnp.minimum(g, na_ref[0] - 1)

    grid_spec = pltpu.PrefetchScalarGridSpec(
        num_scalar_prefetch=2,
        grid=(nb,),
        in_specs=[
            pl.BlockSpec((BM, HIDDEN),
                         lambda g, be, na: (jnp.minimum(g, NB_SH - 1), 0)),
            pl.BlockSpec((BM, HIDDEN),
                         lambda g, be, na: (
                             jnp.clip(g - NB_SH, 0, nbr - 1), 0)),
            pl.BlockSpec((1, HIDDEN, INTER),
                         lambda g, be, na: (be[clampg(g, na)], 0, 0)),
            pl.BlockSpec((1, HIDDEN, INTER),
                         lambda g, be, na: (be[clampg(g, na)], 0, 0)),
            pl.BlockSpec((1, INTER, HIDDEN),
                         lambda g, be, na: (be[clampg(g, na)], 0, 0)),
        ],
        out_specs=pl.BlockSpec((BM, HIDDEN),
                               lambda g, be, na: (clampg(g, na), 0)),
    )
    return pl.pallas_call(
        _mm_body,
        grid_spec=grid_spec,
        out_shape=jax.ShapeDtypeStruct((T + R, HIDDEN), jnp.float32),
    )(block_e, num_active, x_flat, xd, Wg, Wu, Wd)


def kernel(x, gate_w, expert_bias, shared_Wg, shared_Wu, shared_Wd,
           routed_Wg, routed_Wu, routed_Wd):
    Bc, Sc, H = x.shape
    T = Bc * Sc
    A = T * TOP_K
    x_flat = x.reshape(T, H)

    Wg9 = jnp.concatenate([routed_Wg, shared_Wg]).astype(jnp.bfloat16)
    Wu9 = jnp.concatenate([routed_Wu, shared_Wu]).astype(jnp.bfloat16)
    Wd9 = jnp.concatenate([routed_Wd, shared_Wd]).astype(jnp.bfloat16)

    x_fp32 = x.astype(jnp.float32)
    gate_logits = x_fp32 @ gate_w.astype(jnp.float32).T
    scores = jax.nn.sigmoid(gate_logits)
    tie = jnp.arange(N_ROUTED, dtype=jnp.float32) * 1e-6
    scores_for_routing = scores + expert_bias.astype(jnp.float32) + tie
    lanes = jnp.arange(N_ROUTED, dtype=jnp.int32)
    m1 = jnp.max(scores_for_routing, axis=-1, keepdims=True)
    i1 = jnp.min(jnp.where(scores_for_routing == m1, lanes, N_ROUTED),
                 axis=-1, keepdims=True)
    sr2 = jnp.where(lanes == i1, jnp.float32(-1e30), scores_for_routing)
    m2 = jnp.max(sr2, axis=-1, keepdims=True)
    i2 = jnp.min(jnp.where(sr2 == m2, lanes, N_ROUTED),
                 axis=-1, keepdims=True)
    top_k_indices = jnp.concatenate([i1, i2], axis=-1)
    s1 = jnp.sum(jnp.where(lanes == i1, scores, 0.0), axis=-1, keepdims=True)
    s2 = jnp.sum(jnp.where(lanes == i2, scores, 0.0), axis=-1, keepdims=True)
    denom = jnp.maximum(s1 + s2, 1e-9)
    top_k_weights = (jnp.concatenate([s1, s2], axis=-1) / denom).astype(x.dtype)

    idx2 = top_k_indices.reshape(T, TOP_K)
    w2 = top_k_weights.reshape(T, TOP_K)

    e_flat = idx2.reshape(A)
    tok_ids = (jnp.arange(A, dtype=jnp.int32) // TOP_K)
    onehot = (e_flat[:, None] == jnp.arange(N_ROUTED)[None, :]).astype(jnp.int32)
    counts = jnp.sum(onehot, axis=0)
    rank = jnp.sum(jnp.where(onehot == 1, jnp.cumsum(onehot, axis=0) - 1, 0),
                   axis=1)
    padded = ((counts + BM - 1) // BM) * BM
    offs = jnp.concatenate([jnp.zeros((1,), jnp.int32),
                            jnp.cumsum(padded).astype(jnp.int32)])
    pos = offs[e_flat] + rank

    MAXR = A + N_ROUTED * BM
    sorted_ids = jnp.zeros((MAXR,), jnp.int32).at[pos].set(tok_ids)

    NB_RT = MAXR // BM
    NB = NB_SH + NB_RT
    block_start = jnp.arange(NB_RT, dtype=jnp.int32) * BM
    block_e = jnp.clip(
        jnp.sum(block_start[:, None] >= offs[None, 1:], axis=1), 0, N_ROUTED - 1
    ).astype(jnp.int32)
    full_be = jnp.concatenate(
        [jnp.full((NB_SH,), N_ROUTED, jnp.int32), block_e])
    num_active = (NB_SH + offs[N_ROUTED] // BM).reshape(1).astype(jnp.int32)

    xd = jnp.take(x_flat, sorted_ids, axis=0)

    y = _grouped_ffn(x_flat, xd, full_be, num_active, Wg9, Wu9, Wd9, NB)

    p = pos.reshape(T, TOP_K)
    out_flat = (y[:T]
                + w2[:, 0:1] * jnp.take(y, T + p[:, 0], axis=0)
                + w2[:, 1:2] * jnp.take(y, T + p[:, 1], axis=0))
    return out_flat.reshape(Bc, Sc, H), top_k_indices
